# Initial kernel scaffold; baseline (speedup 1.0000x reference)
import jax
import jax.numpy as jnp
from jax.experimental import pallas as pl


def kernel(edge_probs):
    raise NotImplementedError("write your pallas kernel here")


# speedup vs baseline: 1.4584x; 1.4584x over previous
import functools

import jax
import jax.numpy as jnp
import numpy as np
from jax import lax
from jax.experimental import pallas as pl
from jax.experimental.pallas import tpu as pltpu

_BS = 256


def _threefry_bits_u32(x1):
    ks0 = jnp.uint32(0)
    ks1 = jnp.uint32(42)
    ks2 = jnp.uint32(0x1BD11BDA) ^ ks0 ^ ks1
    x0 = jnp.zeros_like(x1) + ks0
    x1 = x1 + ks1

    def rounds(x0, x1, rots):
        for d in rots:
            x0 = x0 + x1
            x1 = (x1 << d) | (x1 >> (32 - d))
            x1 = x1 ^ x0
        return x0, x1

    r_a = (13, 15, 26, 6)
    r_b = (17, 29, 16, 24)
    for i, (a0, a1, rots) in enumerate(
        [(ks1, ks2, r_a), (ks2, ks0, r_b), (ks0, ks1, r_a),
         (ks1, ks2, r_b), (ks2, ks0, r_a)]
    ):
        x0, x1 = rounds(x0, x1, rots)
        x0 = x0 + a0
        x1 = x1 + a1 + jnp.uint32(i + 1)
    return x0 ^ x1


def _body(n, bs, bi_ref, bj_ref, probs_ref, out_ref, scratch_ref):
    p = pl.program_id(0)
    k = pl.program_id(1)
    bi = bi_ref[p]
    bj = bj_ref[p]

    @pl.when(k == 0)
    def _compute():
        rl = lax.broadcasted_iota(jnp.int32, (bs, bs), 0)
        cl = lax.broadcasted_iota(jnp.int32, (bs, bs), 1)
        r = rl + bi * bs
        c = cl + bj * bs
        flat = (r * n + c).astype(jnp.uint32)
        bits = _threefry_bits_u32(flat)
        fbits = (bits >> 9) | jnp.uint32(0x3F800000)
        noise = lax.bitcast_convert_type(fbits, jnp.float32) - 1.0
        bern = (noise < probs_ref[...]).astype(jnp.float32)
        bern_t = bern.T
        diag = bi == bj
        lower = rl > cl
        out_ref[...] = jnp.where(diag & lower, bern_t, bern)
        scratch_ref[...] = jnp.where(diag & jnp.logical_not(lower), bern, bern_t)

    @pl.when(k == 1)
    def _mirror():
        out_ref[...] = scratch_ref[...]


def kernel(edge_probs):
    n = edge_probs.shape[0]
    bs = _BS
    nb = n // bs
    pairs = [(i, j) for i in range(nb) for j in range(i, nb)]
    bi_arr = jnp.asarray(np.array([ij[0] for ij in pairs], dtype=np.int32))
    bj_arr = jnp.asarray(np.array([ij[1] for ij in pairs], dtype=np.int32))
    num_pairs = len(pairs)

    grid_spec = pltpu.PrefetchScalarGridSpec(
        num_scalar_prefetch=2,
        grid=(num_pairs, 2),
        in_specs=[
            pl.BlockSpec((bs, bs), lambda p, k, bi, bj: (bi[p], bj[p])),
        ],
        out_specs=pl.BlockSpec(
            (bs, bs),
            lambda p, k, bi, bj: (
                jnp.where(k == 0, bi[p], bj[p]),
                jnp.where(k == 0, bj[p], bi[p]),
            ),
        ),
        scratch_shapes=[pltpu.VMEM((bs, bs), jnp.float32)],
    )
    return pl.pallas_call(
        functools.partial(_body, n, bs),
        grid_spec=grid_spec,
        out_shape=jax.ShapeDtypeStruct((n, n), jnp.float32),
        compiler_params=pltpu.CompilerParams(
            dimension_semantics=("arbitrary", "arbitrary"),
        ),
    )(bi_arr, bj_arr, edge_probs)
